# baseline (speedup 1.0000x reference)
import functools

import jax
import jax.numpy as jnp
import numpy as np
from jax.experimental import pallas as pl

DIM = 256
NUM_CLASSES = 32
NUM_AGENTS = 512
NUM_STEPS = 4
SLOPE = 0.01
ESLOPE = 0.2
N_NODES = 10000
BLK = 1000
NBLK = N_NODES // BLK


def _ln(x):
    mu = jnp.mean(x, axis=-1, keepdims=True)
    var = jnp.var(x, axis=-1, keepdims=True)
    return (x - mu) / jnp.sqrt(var + 1e-5)


def _lrelu(x, s):
    return jnp.where(x >= 0, x, s * x)


def _time_table(T, d):
    emb = jnp.exp(-(jnp.arange(0, d, 2).astype(jnp.float32) / d * np.log(10000.0)))
    pos = jnp.arange(T).astype(jnp.float32)
    e = pos[:, None] * emb[None, :]
    e = jnp.stack([jnp.sin(e), jnp.cos(e)], axis=-1)
    return e.reshape(T, d)


def _k0_body(x_ref, w_ref, b_ref, o_ref):
    o_ref[...] = (
        jnp.dot(x_ref[...], w_ref[...], preferred_element_type=jnp.float32, precision=jax.lax.Precision.HIGHEST)
        + b_ref[...]
    )


def _k0(x, w, b):
    n, kdim = x.shape
    m = w.shape[1]
    return pl.pallas_call(
        _k0_body,
        grid=(n // BLK,),
        in_specs=[
            pl.BlockSpec((BLK, kdim), lambda i: (i, 0)),
            pl.BlockSpec((kdim, m), lambda i: (0, 0)),
            pl.BlockSpec((1, m), lambda i: (0, 0)),
        ],
        out_specs=pl.BlockSpec((BLK, m), lambda i: (i, 0)),
        out_shape=jax.ShapeDtypeStruct((n, m), jnp.float32),
    )(x, w, b.reshape(1, m))


def _k1_body(node_ref, wq, bq, wk, bk, wm, bm, q_ref, k_ref, m_ref):
    nln = _ln(node_ref[...])
    q = jnp.dot(nln, wq[...], preferred_element_type=jnp.float32, precision=jax.lax.Precision.HIGHEST) + bq[...]
    q_ref[...] = q * (1.0 / 16.0)
    k_ref[...] = jnp.dot(nln, wk[...], preferred_element_type=jnp.float32, precision=jax.lax.Precision.HIGHEST) + bk[...]
    m_ref[...] = _lrelu(
        jnp.dot(nln, wm[...], preferred_element_type=jnp.float32, precision=jax.lax.Precision.HIGHEST) + bm[...], ESLOPE
    )


def _k1(node, wq, bq, wk, bk, wm, bm):
    full = lambda i: (0, 0)
    return pl.pallas_call(
        _k1_body,
        grid=(NBLK,),
        in_specs=[
            pl.BlockSpec((BLK, DIM), lambda i: (i, 0)),
            pl.BlockSpec((DIM, DIM), full),
            pl.BlockSpec((1, DIM), full),
            pl.BlockSpec((DIM, DIM), full),
            pl.BlockSpec((1, DIM), full),
            pl.BlockSpec((DIM, DIM), full),
            pl.BlockSpec((1, DIM), full),
        ],
        out_specs=[
            pl.BlockSpec((BLK, DIM), lambda i: (i, 0)),
            pl.BlockSpec((BLK, DIM), lambda i: (i, 0)),
            pl.BlockSpec((BLK, DIM), lambda i: (i, 0)),
        ],
        out_shape=[
            jax.ShapeDtypeStruct((N_NODES, DIM), jnp.float32),
            jax.ShapeDtypeStruct((N_NODES, DIM), jnp.float32),
            jax.ShapeDtypeStruct((N_NODES, DIM), jnp.float32),
        ],
    )(
        node,
        wq,
        bq.reshape(1, DIM),
        wk,
        bk.reshape(1, DIM),
        wm,
        bm.reshape(1, DIM),
    )


def _k3_body(node_ref, agg_ref, w1, b1, w2, b2, o_ref):
    cu = jnp.concatenate([node_ref[...], agg_ref[...]], axis=-1)
    h = _lrelu(
        jnp.dot(_ln(cu), w1[...], preferred_element_type=jnp.float32, precision=jax.lax.Precision.HIGHEST) + b1[...],
        SLOPE,
    )
    o_ref[...] = (
        node_ref[...]
        + jnp.dot(h, w2[...], preferred_element_type=jnp.float32, precision=jax.lax.Precision.HIGHEST)
        + b2[...]
    )


def _k3(node, agg, w1, b1, w2, b2):
    full = lambda i: (0, 0)
    return pl.pallas_call(
        _k3_body,
        grid=(NBLK,),
        in_specs=[
            pl.BlockSpec((BLK, DIM), lambda i: (i, 0)),
            pl.BlockSpec((BLK, DIM), lambda i: (i, 0)),
            pl.BlockSpec((2 * DIM, 2 * DIM), full),
            pl.BlockSpec((1, 2 * DIM), full),
            pl.BlockSpec((2 * DIM, DIM), full),
            pl.BlockSpec((1, DIM), full),
        ],
        out_specs=pl.BlockSpec((BLK, DIM), lambda i: (i, 0)),
        out_shape=jax.ShapeDtypeStruct((N_NODES, DIM), jnp.float32),
    )(node, agg, w1, b1.reshape(1, 2 * DIM), w2, b2.reshape(1, DIM))


def _k4_body(
    node_ref,
    best_ref,
    pos_ref,
    agent_ref,
    trow_ref,
    wt1,
    bt1,
    wt2,
    bt2,
    wa1,
    ba1,
    wa2,
    ba2,
    wn1,
    bn1,
    wn2,
    bn2,
    agent_o,
    upd_o,
    pos_o,
    nrows_acc,
    nxt_acc,
):
    i = pl.program_id(0)
    pos_col = pos_ref[...]
    rowid = (
        jax.lax.broadcasted_iota(jnp.int32, (NUM_AGENTS, BLK), 1) + i * BLK
    ).astype(jnp.float32)
    oht = (pos_col == rowid).astype(jnp.float32)

    @pl.when(i == 0)
    def _init():
        nrows_acc[...] = jnp.zeros_like(nrows_acc)
        nxt_acc[...] = jnp.zeros_like(nxt_acc)

    nrows_acc[...] += jnp.dot(
        oht, node_ref[...], preferred_element_type=jnp.float32, precision=jax.lax.Precision.HIGHEST
    )
    nxt_acc[...] += jnp.dot(oht, best_ref[...], preferred_element_type=jnp.float32, precision=jax.lax.Precision.HIGHEST)

    @pl.when(i == NBLK - 1)
    def _final():
        temb = (
            jnp.dot(
                _lrelu(
                    jnp.dot(
                        trow_ref[...], wt1[...], preferred_element_type=jnp.float32, precision=jax.lax.Precision.HIGHEST
                    )
                    + bt1[...],
                    SLOPE,
                ),
                wt2[...],
                preferred_element_type=jnp.float32, precision=jax.lax.Precision.HIGHEST,
            )
            + bt2[...]
        )
        nrows = nrows_acc[...]
        agent = agent_ref[...]
        n_at = nrows + temb
        au = jnp.concatenate([agent, n_at], axis=-1)
        h = _lrelu(
            jnp.dot(_ln(au), wa1[...], preferred_element_type=jnp.float32, precision=jax.lax.Precision.HIGHEST)
            + ba1[...],
            SLOPE,
        )
        agent_new = agent + (
            jnp.dot(h, wa2[...], preferred_element_type=jnp.float32, precision=jax.lax.Precision.HIGHEST) + ba2[...]
        )
        agent_o[...] = agent_new
        nu = jnp.concatenate([nrows, agent_new], axis=-1)
        h2 = _lrelu(
            jnp.dot(_ln(nu), wn1[...], preferred_element_type=jnp.float32, precision=jax.lax.Precision.HIGHEST)
            + bn1[...],
            SLOPE,
        )
        upd_o[...] = (
            jnp.dot(h2, wn2[...], preferred_element_type=jnp.float32, precision=jax.lax.Precision.HIGHEST) + bn2[...]
        )
        nxt = nxt_acc[...]
        pos_o[...] = jnp.where(nxt >= float(N_NODES), pos_col, nxt)


def _k4(node, best_f, pos_f, agent, trow, p):
    full = lambda i: (0, 0)
    D2 = 2 * DIM
    out = pl.pallas_call(
        _k4_body,
        grid=(NBLK,),
        in_specs=[
            pl.BlockSpec((BLK, DIM), lambda i: (i, 0)),
            pl.BlockSpec((BLK, 1), lambda i: (i, 0)),
            pl.BlockSpec((NUM_AGENTS, 1), full),
            pl.BlockSpec((NUM_AGENTS, DIM), full),
            pl.BlockSpec((1, DIM), full),
            pl.BlockSpec((DIM, DIM), full),
            pl.BlockSpec((1, DIM), full),
            pl.BlockSpec((DIM, DIM), full),
            pl.BlockSpec((1, DIM), full),
            pl.BlockSpec((D2, D2), full),
            pl.BlockSpec((1, D2), full),
            pl.BlockSpec((D2, DIM), full),
            pl.BlockSpec((1, DIM), full),
            pl.BlockSpec((D2, D2), full),
            pl.BlockSpec((1, D2), full),
            pl.BlockSpec((D2, DIM), full),
            pl.BlockSpec((1, DIM), full),
        ],
        out_specs=[
            pl.BlockSpec((NUM_AGENTS, DIM), full),
            pl.BlockSpec((NUM_AGENTS, DIM), full),
            pl.BlockSpec((NUM_AGENTS, 1), full),
            pl.BlockSpec((NUM_AGENTS, DIM), full),
            pl.BlockSpec((NUM_AGENTS, 1), full),
        ],
        out_shape=[
            jax.ShapeDtypeStruct((NUM_AGENTS, DIM), jnp.float32),
            jax.ShapeDtypeStruct((NUM_AGENTS, DIM), jnp.float32),
            jax.ShapeDtypeStruct((NUM_AGENTS, 1), jnp.float32),
            jax.ShapeDtypeStruct((NUM_AGENTS, DIM), jnp.float32),
            jax.ShapeDtypeStruct((NUM_AGENTS, 1), jnp.float32),
        ],
    )(
        node,
        best_f,
        pos_f,
        agent,
        trow,
        p["Wt1"],
        p["bt1"].reshape(1, DIM),
        p["Wt2"],
        p["bt2"].reshape(1, DIM),
        p["Wa1"],
        p["ba1"].reshape(1, D2),
        p["Wa2"],
        p["ba2"].reshape(1, DIM),
        p["Wn1"],
        p["bn1"].reshape(1, D2),
        p["Wn2"],
        p["bn2"].reshape(1, DIM),
    )
    return out[0], out[1], out[2]


def _k5_body(node_ref, posr_ref, upd_ref, o_ref):
    i = pl.program_id(0)
    rowid = (
        jax.lax.broadcasted_iota(jnp.int32, (BLK, NUM_AGENTS), 0) + i * BLK
    ).astype(jnp.float32)
    oh = (rowid == posr_ref[...]).astype(jnp.float32)
    o_ref[...] = node_ref[...] + jnp.dot(
        oh, upd_ref[...], preferred_element_type=jnp.float32, precision=jax.lax.Precision.HIGHEST
    )


def _k5(node, pos_row, upd):
    full = lambda i: (0, 0)
    return pl.pallas_call(
        _k5_body,
        grid=(NBLK,),
        in_specs=[
            pl.BlockSpec((BLK, DIM), lambda i: (i, 0)),
            pl.BlockSpec((1, NUM_AGENTS), full),
            pl.BlockSpec((NUM_AGENTS, DIM), full),
        ],
        out_specs=pl.BlockSpec((BLK, DIM), lambda i: (i, 0)),
        out_shape=jax.ShapeDtypeStruct((N_NODES, DIM), jnp.float32),
    )(node, pos_row, upd)


def _k6_body(node_ref, agent_ref, wo, bo, out_ref, acc):
    i = pl.program_id(0)

    @pl.when(i == 0)
    def _init():
        acc[...] = jnp.zeros_like(acc)

    acc[...] += jnp.sum(node_ref[...], axis=0, keepdims=True)

    @pl.when(i == NBLK - 1)
    def _final():
        pooled = acc[...] / float(N_NODES) + jnp.mean(
            agent_ref[...], axis=0, keepdims=True
        )
        out_ref[...] = (
            jnp.dot(pooled, wo[...], preferred_element_type=jnp.float32, precision=jax.lax.Precision.HIGHEST) + bo[...]
        )


def _k6(node, agent, wo, bo):
    full = lambda i: (0, 0)
    out = pl.pallas_call(
        _k6_body,
        grid=(NBLK,),
        in_specs=[
            pl.BlockSpec((BLK, DIM), lambda i: (i, 0)),
            pl.BlockSpec((NUM_AGENTS, DIM), full),
            pl.BlockSpec((DIM, NUM_CLASSES), full),
            pl.BlockSpec((1, NUM_CLASSES), full),
        ],
        out_specs=[
            pl.BlockSpec((1, NUM_CLASSES), full),
            pl.BlockSpec((1, DIM), full),
        ],
        out_shape=[
            jax.ShapeDtypeStruct((1, NUM_CLASSES), jnp.float32),
            jax.ShapeDtypeStruct((1, DIM), jnp.float32),
        ],
    )(node, agent, wo, bo.reshape(1, NUM_CLASSES))
    return out[0]


def _edge_stage(q, k, msg, src, dst, N):
    score = _lrelu(jnp.sum(q[src] * k[dst], axis=-1), ESLOPE)
    m = jax.ops.segment_max(score, src, num_segments=N)
    ex = jnp.exp(score - m[src])
    den = jax.ops.segment_sum(ex, src, num_segments=N) + 1e-9
    alpha = ex / den[src]
    agg = jax.ops.segment_sum(alpha[:, None] * msg[dst], src, num_segments=N)
    is_max = score >= m[src] - 1e-6
    cand = jnp.where(is_max, dst, N)
    best = jax.ops.segment_min(cand, src, num_segments=N)
    return agg, best


def kernel(x, params, edge_index, batch):
    p = params
    src = edge_index[0]
    dst = edge_index[1]
    N = x.shape[0]
    A = NUM_AGENTS

    b0 = (p["b_in"] + p["node_mem_init"]).astype(jnp.float32)
    node = _k0(x, p["W_in"], b0)
    agent = p["agent_emb"]
    pos_f = (jnp.arange(A, dtype=jnp.float32) % N).reshape(A, 1)
    ttab = _time_table(NUM_STEPS + 1, DIM)

    for t in range(NUM_STEPS):
        q, k, msg = _k1(
            node, p["Wq"], p["bq"], p["Wk"], p["bk"], p["Wmsg"], p["bmsg"]
        )
        agg, best = _edge_stage(q, k, msg, src, dst, N)
        node = _k3(node, agg, p["Wc1"], p["bc1"], p["Wc2"], p["bc2"])
        best_f = best.astype(jnp.float32).reshape(N, 1)
        trow = ttab[t].reshape(1, DIM)
        agent, upd, pos_new = _k4(node, best_f, pos_f, agent, trow, p)
        node = _k5(node, pos_f.reshape(1, A), upd)
        pos_f = pos_new

    return _k6(node, agent, p["Wo"], p["bo"])


# speedup vs baseline: 1.0000x; 1.0000x over previous
import functools

import jax
import jax.numpy as jnp
import numpy as np
from jax import lax
from jax.experimental import pallas as pl
from jax.experimental.pallas import tpu as pltpu
from jax.experimental.pallas import tpu_sc as plsc

DIM = 256
NUM_CLASSES = 32
NUM_AGENTS = 512
NUM_STEPS = 4
SLOPE = 0.01
ESLOPE = 0.2
N_NODES = 10000
BLK = 1000
NBLK = N_NODES // BLK


def _ln(x):
    mu = jnp.mean(x, axis=-1, keepdims=True)
    var = jnp.var(x, axis=-1, keepdims=True)
    return (x - mu) / jnp.sqrt(var + 1e-5)


def _lrelu(x, s):
    return jnp.where(x >= 0, x, s * x)


def _time_table(T, d):
    emb = jnp.exp(-(jnp.arange(0, d, 2).astype(jnp.float32) / d * np.log(10000.0)))
    pos = jnp.arange(T).astype(jnp.float32)
    e = pos[:, None] * emb[None, :]
    e = jnp.stack([jnp.sin(e), jnp.cos(e)], axis=-1)
    return e.reshape(T, d)


def _k0_body(x_ref, w_ref, b_ref, o_ref):
    o_ref[...] = (
        jnp.dot(x_ref[...], w_ref[...], preferred_element_type=jnp.float32, precision=jax.lax.Precision.HIGHEST)
        + b_ref[...]
    )


def _k0(x, w, b):
    n, kdim = x.shape
    m = w.shape[1]
    return pl.pallas_call(
        _k0_body,
        grid=(n // BLK,),
        in_specs=[
            pl.BlockSpec((BLK, kdim), lambda i: (i, 0)),
            pl.BlockSpec((kdim, m), lambda i: (0, 0)),
            pl.BlockSpec((1, m), lambda i: (0, 0)),
        ],
        out_specs=pl.BlockSpec((BLK, m), lambda i: (i, 0)),
        out_shape=jax.ShapeDtypeStruct((n, m), jnp.float32),
    )(x, w, b.reshape(1, m))


def _k1_body(node_ref, wq, bq, wk, bk, wm, bm, q_ref, k_ref, m_ref):
    nln = _ln(node_ref[...])
    q = jnp.dot(nln, wq[...], preferred_element_type=jnp.float32, precision=jax.lax.Precision.HIGHEST) + bq[...]
    q_ref[...] = q * (1.0 / 16.0)
    k_ref[...] = jnp.dot(nln, wk[...], preferred_element_type=jnp.float32, precision=jax.lax.Precision.HIGHEST) + bk[...]
    m_ref[...] = _lrelu(
        jnp.dot(nln, wm[...], preferred_element_type=jnp.float32, precision=jax.lax.Precision.HIGHEST) + bm[...], ESLOPE
    )


def _k1(node, wq, bq, wk, bk, wm, bm):
    full = lambda i: (0, 0)
    return pl.pallas_call(
        _k1_body,
        grid=(NBLK,),
        in_specs=[
            pl.BlockSpec((BLK, DIM), lambda i: (i, 0)),
            pl.BlockSpec((DIM, DIM), full),
            pl.BlockSpec((1, DIM), full),
            pl.BlockSpec((DIM, DIM), full),
            pl.BlockSpec((1, DIM), full),
            pl.BlockSpec((DIM, DIM), full),
            pl.BlockSpec((1, DIM), full),
        ],
        out_specs=[
            pl.BlockSpec((BLK, DIM), lambda i: (i, 0)),
            pl.BlockSpec((BLK, DIM), lambda i: (i, 0)),
            pl.BlockSpec((BLK, DIM), lambda i: (i, 0)),
        ],
        out_shape=[
            jax.ShapeDtypeStruct((N_NODES, DIM), jnp.float32),
            jax.ShapeDtypeStruct((N_NODES, DIM), jnp.float32),
            jax.ShapeDtypeStruct((N_NODES, DIM), jnp.float32),
        ],
    )(
        node,
        wq,
        bq.reshape(1, DIM),
        wk,
        bk.reshape(1, DIM),
        wm,
        bm.reshape(1, DIM),
    )


def _k3_body(node_ref, agg_ref, w1, b1, w2, b2, o_ref):
    cu = jnp.concatenate([node_ref[...], agg_ref[...]], axis=-1)
    h = _lrelu(
        jnp.dot(_ln(cu), w1[...], preferred_element_type=jnp.float32, precision=jax.lax.Precision.HIGHEST) + b1[...],
        SLOPE,
    )
    o_ref[...] = (
        node_ref[...]
        + jnp.dot(h, w2[...], preferred_element_type=jnp.float32, precision=jax.lax.Precision.HIGHEST)
        + b2[...]
    )


def _k3(node, agg, w1, b1, w2, b2):
    full = lambda i: (0, 0)
    return pl.pallas_call(
        _k3_body,
        grid=(NBLK,),
        in_specs=[
            pl.BlockSpec((BLK, DIM), lambda i: (i, 0)),
            pl.BlockSpec((BLK, DIM), lambda i: (i, 0)),
            pl.BlockSpec((2 * DIM, 2 * DIM), full),
            pl.BlockSpec((1, 2 * DIM), full),
            pl.BlockSpec((2 * DIM, DIM), full),
            pl.BlockSpec((1, DIM), full),
        ],
        out_specs=pl.BlockSpec((BLK, DIM), lambda i: (i, 0)),
        out_shape=jax.ShapeDtypeStruct((N_NODES, DIM), jnp.float32),
    )(node, agg, w1, b1.reshape(1, 2 * DIM), w2, b2.reshape(1, DIM))


def _k4_body(
    node_ref,
    best_ref,
    pos_ref,
    agent_ref,
    trow_ref,
    wt1,
    bt1,
    wt2,
    bt2,
    wa1,
    ba1,
    wa2,
    ba2,
    wn1,
    bn1,
    wn2,
    bn2,
    agent_o,
    upd_o,
    pos_o,
    nrows_acc,
    nxt_acc,
):
    i = pl.program_id(0)
    pos_col = pos_ref[...]
    rowid = (
        jax.lax.broadcasted_iota(jnp.int32, (NUM_AGENTS, BLK), 1) + i * BLK
    ).astype(jnp.float32)
    oht = (pos_col == rowid).astype(jnp.float32)

    @pl.when(i == 0)
    def _init():
        nrows_acc[...] = jnp.zeros_like(nrows_acc)
        nxt_acc[...] = jnp.zeros_like(nxt_acc)

    nrows_acc[...] += jnp.dot(
        oht, node_ref[...], preferred_element_type=jnp.float32, precision=jax.lax.Precision.HIGHEST
    )
    nxt_acc[...] += jnp.dot(oht, best_ref[...], preferred_element_type=jnp.float32, precision=jax.lax.Precision.HIGHEST)

    @pl.when(i == NBLK - 1)
    def _final():
        temb = (
            jnp.dot(
                _lrelu(
                    jnp.dot(
                        trow_ref[...], wt1[...], preferred_element_type=jnp.float32, precision=jax.lax.Precision.HIGHEST
                    )
                    + bt1[...],
                    SLOPE,
                ),
                wt2[...],
                preferred_element_type=jnp.float32, precision=jax.lax.Precision.HIGHEST,
            )
            + bt2[...]
        )
        nrows = nrows_acc[...]
        agent = agent_ref[...]
        n_at = nrows + temb
        au = jnp.concatenate([agent, n_at], axis=-1)
        h = _lrelu(
            jnp.dot(_ln(au), wa1[...], preferred_element_type=jnp.float32, precision=jax.lax.Precision.HIGHEST)
            + ba1[...],
            SLOPE,
        )
        agent_new = agent + (
            jnp.dot(h, wa2[...], preferred_element_type=jnp.float32, precision=jax.lax.Precision.HIGHEST) + ba2[...]
        )
        agent_o[...] = agent_new
        nu = jnp.concatenate([nrows, agent_new], axis=-1)
        h2 = _lrelu(
            jnp.dot(_ln(nu), wn1[...], preferred_element_type=jnp.float32, precision=jax.lax.Precision.HIGHEST)
            + bn1[...],
            SLOPE,
        )
        upd_o[...] = (
            jnp.dot(h2, wn2[...], preferred_element_type=jnp.float32, precision=jax.lax.Precision.HIGHEST) + bn2[...]
        )
        nxt = nxt_acc[...]
        pos_o[...] = jnp.where(nxt >= float(N_NODES), pos_col, nxt)


def _k4(node, best_f, pos_f, agent, trow, p):
    full = lambda i: (0, 0)
    D2 = 2 * DIM
    out = pl.pallas_call(
        _k4_body,
        grid=(NBLK,),
        in_specs=[
            pl.BlockSpec((BLK, DIM), lambda i: (i, 0)),
            pl.BlockSpec((BLK, 1), lambda i: (i, 0)),
            pl.BlockSpec((NUM_AGENTS, 1), full),
            pl.BlockSpec((NUM_AGENTS, DIM), full),
            pl.BlockSpec((1, DIM), full),
            pl.BlockSpec((DIM, DIM), full),
            pl.BlockSpec((1, DIM), full),
            pl.BlockSpec((DIM, DIM), full),
            pl.BlockSpec((1, DIM), full),
            pl.BlockSpec((D2, D2), full),
            pl.BlockSpec((1, D2), full),
            pl.BlockSpec((D2, DIM), full),
            pl.BlockSpec((1, DIM), full),
            pl.BlockSpec((D2, D2), full),
            pl.BlockSpec((1, D2), full),
            pl.BlockSpec((D2, DIM), full),
            pl.BlockSpec((1, DIM), full),
        ],
        out_specs=[
            pl.BlockSpec((NUM_AGENTS, DIM), full),
            pl.BlockSpec((NUM_AGENTS, DIM), full),
            pl.BlockSpec((NUM_AGENTS, 1), full),
            pl.BlockSpec((NUM_AGENTS, DIM), full),
            pl.BlockSpec((NUM_AGENTS, 1), full),
        ],
        out_shape=[
            jax.ShapeDtypeStruct((NUM_AGENTS, DIM), jnp.float32),
            jax.ShapeDtypeStruct((NUM_AGENTS, DIM), jnp.float32),
            jax.ShapeDtypeStruct((NUM_AGENTS, 1), jnp.float32),
            jax.ShapeDtypeStruct((NUM_AGENTS, DIM), jnp.float32),
            jax.ShapeDtypeStruct((NUM_AGENTS, 1), jnp.float32),
        ],
    )(
        node,
        best_f,
        pos_f,
        agent,
        trow,
        p["Wt1"],
        p["bt1"].reshape(1, DIM),
        p["Wt2"],
        p["bt2"].reshape(1, DIM),
        p["Wa1"],
        p["ba1"].reshape(1, D2),
        p["Wa2"],
        p["ba2"].reshape(1, DIM),
        p["Wn1"],
        p["bn1"].reshape(1, D2),
        p["Wn2"],
        p["bn2"].reshape(1, DIM),
    )
    return out[0], out[1], out[2]


def _k5_body(node_ref, posr_ref, upd_ref, o_ref):
    i = pl.program_id(0)
    rowid = (
        jax.lax.broadcasted_iota(jnp.int32, (BLK, NUM_AGENTS), 0) + i * BLK
    ).astype(jnp.float32)
    oh = (rowid == posr_ref[...]).astype(jnp.float32)
    o_ref[...] = node_ref[...] + jnp.dot(
        oh, upd_ref[...], preferred_element_type=jnp.float32, precision=jax.lax.Precision.HIGHEST
    )


def _k5(node, pos_row, upd):
    full = lambda i: (0, 0)
    return pl.pallas_call(
        _k5_body,
        grid=(NBLK,),
        in_specs=[
            pl.BlockSpec((BLK, DIM), lambda i: (i, 0)),
            pl.BlockSpec((1, NUM_AGENTS), full),
            pl.BlockSpec((NUM_AGENTS, DIM), full),
        ],
        out_specs=pl.BlockSpec((BLK, DIM), lambda i: (i, 0)),
        out_shape=jax.ShapeDtypeStruct((N_NODES, DIM), jnp.float32),
    )(node, pos_row, upd)


def _k6_body(node_ref, agent_ref, wo, bo, out_ref, acc):
    i = pl.program_id(0)

    @pl.when(i == 0)
    def _init():
        acc[...] = jnp.zeros_like(acc)

    acc[...] += jnp.sum(node_ref[...], axis=0, keepdims=True)

    @pl.when(i == NBLK - 1)
    def _final():
        pooled = acc[...] / float(N_NODES) + jnp.mean(
            agent_ref[...], axis=0, keepdims=True
        )
        out_ref[...] = (
            jnp.dot(pooled, wo[...], preferred_element_type=jnp.float32, precision=jax.lax.Precision.HIGHEST) + bo[...]
        )


def _k6(node, agent, wo, bo):
    full = lambda i: (0, 0)
    out = pl.pallas_call(
        _k6_body,
        grid=(NBLK,),
        in_specs=[
            pl.BlockSpec((BLK, DIM), lambda i: (i, 0)),
            pl.BlockSpec((NUM_AGENTS, DIM), full),
            pl.BlockSpec((DIM, NUM_CLASSES), full),
            pl.BlockSpec((1, NUM_CLASSES), full),
        ],
        out_specs=[
            pl.BlockSpec((1, NUM_CLASSES), full),
            pl.BlockSpec((1, DIM), full),
        ],
        out_shape=[
            jax.ShapeDtypeStruct((1, NUM_CLASSES), jnp.float32),
            jax.ShapeDtypeStruct((1, DIM), jnp.float32),
        ],
    )(node, agent, wo, bo.reshape(1, NUM_CLASSES))
    return out[0]


NW = 32
NPW = 320
CH = 64
N_PAD = NW * NPW
NEG_INF = float("-inf")


def _iota16():
    return lax.broadcasted_iota(jnp.int32, (16,), 0)


_LOG2E = 1.4426950408889634
_LN2_HI = 0.6931471824645996
_LN2_LO = -1.904654323148236e-09


def _exp_acc(x):
    x = jnp.minimum(jnp.maximum(x, -80.0), 80.0)
    kf = x * _LOG2E
    kf = jnp.where(kf >= 0, kf + 0.5, kf - 0.5)
    ki = kf.astype(jnp.int32)
    kr = ki.astype(jnp.float32)
    r = (x - kr * _LN2_HI) - kr * _LN2_LO
    p = 1.0 / 720.0
    p = p * r + 1.0 / 120.0
    p = p * r + 1.0 / 24.0
    p = p * r + 1.0 / 6.0
    p = p * r + 0.5
    p = p * r + 1.0
    p = p * r + 1.0
    scale = plsc.bitcast((ki + 127) << 23, jnp.float32)
    return p * scale


def _recip(d):
    r = 1.0 / d
    r = r * (2.0 - d * r)
    return r


def _vread(ref, idx):
    v = plsc.load_gather(ref, [jnp.full((16,), idx, jnp.int32)])
    return jnp.max(v)


def _swrite(ref, idx, val):
    plsc.store_scatter(
        ref,
        [jnp.full((16,), idx, jnp.int32)],
        jnp.full((16,), val),
        mask=_iota16() == 0,
    )


def _edge_body(
    q_hbm,
    k_hbm,
    msg_hbm,
    src_hbm,
    dst_hbm,
    rs_hbm,
    agg_hbm,
    best_hbm,
    sco_hbm,
    aggbuf,
    rows_a,
    rows_b,
    rs_v,
    src_v,
    dst_v,
    sco_v,
    m_buf,
    den_buf,
    best_buf,
    sem_a,
    sem_b,
):
    wid = lax.axis_index("s") * 2 + lax.axis_index("c")
    n0 = pl.multiple_of(wid * NPW, 8)
    pltpu.sync_copy(rs_hbm.at[pl.ds(n0, NPW + 16)], rs_v)
    nv = jnp.minimum(N_NODES - n0, NPW)
    e0 = _vread(rs_v, 0)
    e1 = _vread(rs_v, nv)
    ea = pl.multiple_of(e0 - lax.rem(e0, 8), 8)
    nch = (e1 - ea + (CH - 1)) // CH

    zero16 = jnp.zeros((16,), jnp.float32)
    ninf16 = jnp.full((16,), NEG_INF, jnp.float32)
    nbig16 = jnp.full((16,), float(N_NODES), jnp.float32)
    for j in range(NPW // 16):
        m_buf[pl.ds(j * 16, 16)] = ninf16
        den_buf[pl.ds(j * 16, 16)] = zero16
        best_buf[pl.ds(j * 16, 16)] = nbig16

    def zrow(j, _):
        for c in range(DIM // 16):
            aggbuf[j, pl.ds(c * 16, 16)] = zero16
        return 0

    lax.fori_loop(0, NPW, zrow, 0)


    def a_chunk(c, car):
        cur_ln, cur_m = car
        base = pl.multiple_of(ea + c * CH, 8)
        pltpu.sync_copy(src_hbm.at[pl.ds(base, CH)], src_v)
        pltpu.sync_copy(dst_hbm.at[pl.ds(base, CH)], dst_v)
        ca = pltpu.async_copy(q_hbm.at[src_v], rows_a, sem_a)
        cb = pltpu.async_copy(k_hbm.at[dst_v], rows_b, sem_b)
        ca.wait()
        cb.wait()
        for g in range(4):
            rid = _iota16() + (g * 16)

            parts = []
            for pj in range(8):

                def dbody(dc, acc, pj=pj):
                    out = acc
                    for dd in range(8):
                        dspl = jnp.full(
                            (16,), pj * 32 + dc * 8 + dd, jnp.int32
                        )
                        qv = plsc.load_gather(rows_a, [rid, dspl])
                        kv = plsc.load_gather(rows_b, [rid, dspl])
                        out = out + qv * kv
                    return out

                parts.append(lax.fori_loop(0, 4, dbody, zero16))
            acc = (
                (parts[0] + parts[1]) + (parts[2] + parts[3])
            ) + ((parts[4] + parts[5]) + (parts[6] + parts[7]))
            sc = jnp.where(acc >= 0.0, acc, ESLOPE * acc)
            sco_v[pl.ds(g * 16, 16)] = sc
            gl = base + g * 16 + _iota16()
            valid = (gl >= e0) & (gl < e1)
            lnz = jnp.where(valid, src_v[pl.ds(g * 16, 16)] - n0, -1)
            scm = jnp.where(valid, sc, NEG_INF)

            def mupd(jj, car2):
                ln2, m2 = car2
                lj = jnp.max(jnp.where(_iota16() == jj, lnz, -1))
                sj = jnp.max(jnp.where(_iota16() == jj, scm, NEG_INF))
                same = (lj == ln2) | (lj < 0)

                @pl.when(jnp.logical_not(same) & (ln2 >= 0))
                def _flush():
                    _swrite(m_buf, ln2, m2)

                m3 = jnp.where(same, jnp.maximum(m2, sj), sj)
                ln3 = jnp.where(lj < 0, ln2, lj)
                return ln3, m3

            cur_ln, cur_m = lax.fori_loop(0, 16, mupd, (cur_ln, cur_m))
        pltpu.sync_copy(sco_v, sco_hbm.at[pl.ds(base, CH)])
        return cur_ln, cur_m

    cur_ln, cur_m = lax.fori_loop(
        0, nch, a_chunk, (jnp.int32(-1), jnp.float32(NEG_INF))
    )

    @pl.when(cur_ln >= 0)
    def _mflush():
        _swrite(m_buf, cur_ln, cur_m)

    def b_chunk(c, car):
        cur_ln, cur_den, cur_best = car
        base = pl.multiple_of(ea + c * CH, 8)
        pltpu.sync_copy(src_hbm.at[pl.ds(base, CH)], src_v)
        pltpu.sync_copy(dst_hbm.at[pl.ds(base, CH)], dst_v)
        pltpu.sync_copy(sco_hbm.at[pl.ds(base, CH)], sco_v)
        for g in range(4):
            gl = base + g * 16 + _iota16()
            valid = (gl >= e0) & (gl < e1)
            ln_raw = src_v[pl.ds(g * 16, 16)] - n0
            lnz = jnp.where(valid, ln_raw, -1)
            lnc = jnp.maximum(lnz, 0)
            sc = sco_v[pl.ds(g * 16, 16)]
            mv = plsc.load_gather(m_buf, [lnc])
            exm = jnp.where(valid, _exp_acc(sc - mv), 0.0)
            dstf = dst_v[pl.ds(g * 16, 16)].astype(jnp.float32)
            cand = jnp.where(
                valid & (sc >= mv - 1e-6), dstf, float(N_NODES)
            )
            candm = jnp.where(valid, cand, jnp.inf)

            def supd(jj, car2):
                ln2, den2, best2 = car2
                lj = jnp.max(jnp.where(_iota16() == jj, lnz, -1))
                ej = jnp.max(jnp.where(_iota16() == jj, exm, 0.0))
                cj = jnp.min(jnp.where(_iota16() == jj, candm, jnp.inf))
                same = (lj == ln2) | (lj < 0)

                @pl.when(jnp.logical_not(same) & (ln2 >= 0))
                def _flush():
                    _swrite(den_buf, ln2, den2)
                    _swrite(best_buf, ln2, best2)

                den3 = jnp.where(same, den2 + ej, ej)
                best3 = jnp.where(same, jnp.minimum(best2, cj), cj)
                ln3 = jnp.where(lj < 0, ln2, lj)
                return ln3, den3, best3

            cur_ln, cur_den, cur_best = lax.fori_loop(
                0, 16, supd, (cur_ln, cur_den, cur_best)
            )
        return cur_ln, cur_den, cur_best

    cur_ln, cur_den, cur_best = lax.fori_loop(
        0,
        nch,
        b_chunk,
        (jnp.int32(-1), jnp.float32(0.0), jnp.float32(float(N_NODES))),
    )

    @pl.when(cur_ln >= 0)
    def _dflush():
        _swrite(den_buf, cur_ln, cur_den)
        _swrite(best_buf, cur_ln, cur_best)

    def c_chunk(c, car):
        cur_ln = car[0]
        accs = car[1:]
        base = pl.multiple_of(ea + c * CH, 8)
        pltpu.sync_copy(src_hbm.at[pl.ds(base, CH)], src_v)
        pltpu.sync_copy(dst_hbm.at[pl.ds(base, CH)], dst_v)
        pltpu.sync_copy(sco_hbm.at[pl.ds(base, CH)], sco_v)
        pltpu.async_copy(msg_hbm.at[dst_v], rows_a, sem_a).wait()
        for g in range(4):
            gl = base + g * 16 + _iota16()
            valid = (gl >= e0) & (gl < e1)
            ln_raw = src_v[pl.ds(g * 16, 16)] - n0
            lnz = jnp.where(valid, ln_raw, -1)
            lnc = jnp.maximum(lnz, 0)
            sc = sco_v[pl.ds(g * 16, 16)]
            mv = plsc.load_gather(m_buf, [lnc])
            dv = plsc.load_gather(den_buf, [lnc])
            alm = jnp.where(valid, _exp_acc(sc - mv) * _recip(dv + 1e-9), 0.0)

            def aupd(jj, car2, g=g):
                ln2 = car2[0]
                acc2 = car2[1:]
                lj = jnp.max(jnp.where(_iota16() == jj, lnz, -1))
                aj = jnp.max(jnp.where(_iota16() == jj, alm, 0.0))
                row = g * 16 + jj
                same = (lj == ln2) | (lj < 0)

                @pl.when(jnp.logical_not(same) & (ln2 >= 0))
                def _flush():
                    for cc in range(DIM // 16):
                        aggbuf[ln2, pl.ds(cc * 16, 16)] = acc2[cc]

                keep = jnp.where(same, 1.0, 0.0)
                acc3 = tuple(
                    acc2[cc] * keep
                    + aj * rows_a[row, pl.ds(cc * 16, 16)]
                    for cc in range(DIM // 16)
                )
                ln3 = jnp.where(lj < 0, ln2, lj)
                return (ln3,) + acc3

            car_in = (cur_ln,) + accs
            car_out = lax.fori_loop(0, 16, aupd, car_in)
            cur_ln = car_out[0]
            accs = car_out[1:]
        return (cur_ln,) + accs

    cinit = (jnp.int32(-1),) + tuple(zero16 for _ in range(DIM // 16))
    cout = lax.fori_loop(0, nch, c_chunk, cinit)
    cur_ln = cout[0]

    @pl.when(cur_ln >= 0)
    def _aflush():
        for cc in range(DIM // 16):
            aggbuf[cur_ln, pl.ds(cc * 16, 16)] = cout[1 + cc]

    pltpu.sync_copy(aggbuf, agg_hbm.at[pl.ds(n0, NPW)])
    pltpu.sync_copy(best_buf, best_hbm.at[pl.ds(n0, NPW)])


def _edge_sc(q_pad, k, msg, src_p, dst_p, rs_pad):
    mesh = plsc.VectorSubcoreMesh(
        core_axis_name="c", subcore_axis_name="s", num_cores=2, num_subcores=16
    )
    kern = pl.kernel(
        _edge_body,
        out_type=[
            jax.ShapeDtypeStruct((N_PAD, DIM), jnp.float32),
            jax.ShapeDtypeStruct((N_PAD,), jnp.float32),
            jax.ShapeDtypeStruct((src_p.shape[0],), jnp.float32),
        ],
        mesh=mesh,
        compiler_params=pltpu.CompilerParams(needs_layout_passes=False),
        scratch_types=[
            pltpu.VMEM((NPW, DIM), jnp.float32),
            pltpu.VMEM((CH, DIM), jnp.float32),
            pltpu.VMEM((CH, DIM), jnp.float32),
            pltpu.VMEM((NPW + 16,), jnp.int32),
            pltpu.VMEM((CH,), jnp.int32),
            pltpu.VMEM((CH,), jnp.int32),
            pltpu.VMEM((CH,), jnp.float32),
            pltpu.VMEM((NPW,), jnp.float32),
            pltpu.VMEM((NPW,), jnp.float32),
            pltpu.VMEM((NPW,), jnp.float32),
            pltpu.SemaphoreType.DMA,
            pltpu.SemaphoreType.DMA,
        ],
    )
    agg, best, _ = kern(q_pad, k, msg, src_p, dst_p, rs_pad)
    return agg, best


def kernel(x, params, edge_index, batch):
    p = params
    src = edge_index[0]
    dst = edge_index[1]
    N = x.shape[0]
    A = NUM_AGENTS

    b0 = (p["b_in"] + p["node_mem_init"]).astype(jnp.float32)
    node = _k0(x, p["W_in"], b0)
    agent = p["agent_emb"]
    pos_f = (jnp.arange(A, dtype=jnp.float32) % N).reshape(A, 1)
    ttab = _time_table(NUM_STEPS + 1, DIM)

    perm = jnp.argsort(src)
    src_s = src[perm].astype(jnp.int32)
    dst_s = dst[perm].astype(jnp.int32)
    rs_pad = jnp.searchsorted(
        src_s, jnp.arange(N_PAD + 16, dtype=jnp.int32)
    ).astype(jnp.int32)
    zpad = jnp.zeros((128,), jnp.int32)
    src_p = jnp.concatenate([src_s, zpad])
    dst_p = jnp.concatenate([dst_s, zpad])

    for t in range(NUM_STEPS):
        q, k, msg = _k1(
            node, p["Wq"], p["bq"], p["Wk"], p["bk"], p["Wmsg"], p["bmsg"]
        )
        score = _lrelu(jnp.sum(q[src] * k[dst], axis=-1), ESLOPE)
        mseg = jax.ops.segment_max(score, src, num_segments=N)
        ex = jnp.exp(score - mseg[src])
        den = jax.ops.segment_sum(ex, src, num_segments=N) + 1e-9
        alpha = ex / den[src]
        agg = jax.ops.segment_sum(alpha[:, None] * msg[dst], src, num_segments=N)
        node = _k3(node, agg, p["Wc1"], p["bc1"], p["Wc2"], p["bc2"])
        best_x = jax.ops.segment_min(
            jnp.where(score >= mseg[src] - 1e-6, dst, N), src, num_segments=N
        )
        best_f = best_x.astype(jnp.float32).reshape(N, 1)
        trow = ttab[t].reshape(1, DIM)
        agent, upd, pos_new = _k4(node, best_f, pos_f, agent, trow, p)
        node = _k5(node, pos_f.reshape(1, A), upd)
        pos_f = pos_new

    return _k6(node, agent, p["Wo"], p["bo"])
